# baseline (speedup 1.0000x reference)
import jax
import jax.numpy as jnp
from jax.experimental import pallas as pl
from jax.experimental.pallas import tpu as pltpu

_N = 10000
_M = 10000
_BQ = 1000
_M_PAD = 10112
_FAR = 1e9


def _knn_flow_kernel(q_ref, rpt_ref, rb_ref, rft_ref, out_ref):
    q = q_ref[...]
    qx, qy, qz = q[:, 0:1], q[:, 1:2], q[:, 2:3]
    rx, ry, rz = rpt_ref[0:1, :], rpt_ref[1:2, :], rpt_ref[2:3, :]

    q2 = qx * qx + qy * qy + qz * qz
    r2 = rx * rx + ry * ry + rz * rz
    qrm2 = jax.lax.dot_general(
        (q * -2.0).astype(jnp.bfloat16), rb_ref[...],
        dimension_numbers=(((1,), (0,)), ((), ())),
        preferred_element_type=jnp.float32,
    )

    big = jnp.float32(jnp.inf)
    zero = jnp.float32(0.0)
    ngrp = _M_PAD // 128
    shape128 = (qrm2.shape[0], 128)
    V1 = jnp.full(shape128, big)
    V2 = jnp.full(shape128, big)
    V3 = jnp.full(shape128, big)
    G1 = jnp.zeros(shape128)
    G2 = jnp.zeros(shape128)
    G3 = jnp.zeros(shape128)
    for g in range(ngrp):
        sl = slice(g * 128, (g + 1) * 128)
        v = jnp.sqrt(jnp.maximum((q2 + qrm2[:, sl]) + r2[:, sl], 0.0))
        gf = jnp.float32(g)
        lt1 = v < V1
        nV1 = jnp.where(lt1, v, V1)
        d1 = jnp.where(lt1, V1, v)
        nG1 = jnp.where(lt1, gf, G1)
        dg1 = jnp.where(lt1, G1, gf)
        lt2 = d1 < V2
        nV2 = jnp.where(lt2, d1, V2)
        d2c = jnp.where(lt2, V2, d1)
        nG2 = jnp.where(lt2, dg1, G2)
        dg2 = jnp.where(lt2, G2, dg1)
        lt3 = d2c < V3
        V3 = jnp.where(lt3, d2c, V3)
        G3 = jnp.where(lt3, dg2, G3)
        V1, V2, G1, G2 = nV1, nV2, nG1, nG2

    iotaL = jax.lax.broadcasted_iota(jnp.int32, (1, 128), 1).astype(jnp.float32)
    bigidx = jnp.float32(1e9)

    def _extract(V1, V2, V3, G1, G2, G3):
        m = jnp.min(V1, axis=1, keepdims=True)
        gidx = G1 * 128.0 + iotaL
        i = jnp.min(jnp.where(V1 == m, gidx, bigidx), axis=1, keepdims=True)
        l = i - 128.0 * jnp.floor(i * (1.0 / 128.0))
        eqL = iotaL == l
        V1 = jnp.where(eqL, V2, V1)
        G1 = jnp.where(eqL, G2, G1)
        V2 = jnp.where(eqL, V3, V2)
        G2 = jnp.where(eqL, G3, G2)
        return m, i, V1, V2, G1, G2

    m1, i1, V1, V2, G1, G2 = _extract(V1, V2, V3, G1, G2, G3)
    m2, i2, V1, V2, G1, G2 = _extract(V1, V2, V3, G1, G2, G3)
    m3, i3, _, _, _, _ = _extract(V1, V2, V3, G1, G2, G3)

    iota128 = jax.lax.broadcasted_iota(jnp.int32, (1, 128), 1).astype(jnp.float32)
    rfa = rft_ref[...]
    zero = jnp.float32(0.0)

    def _pick(i):
        g = jnp.floor(i * (1.0 / 128.0))
        l = i - g * 128.0
        oh = jnp.where(iota128 == g, 1.0, zero)
        p = jax.lax.dot_general(
            oh, rfa, dimension_numbers=(((1,), (0,)), ((), ())),
            preferred_element_type=jnp.float32,
        )
        eql = iota128 == l
        return p, eql

    p1, el1 = _pick(i1)
    p2, el2 = _pick(i2)
    p3, el3 = _pick(i3)

    inv1 = 1.0 / (m1 + 1e-8)
    inv2 = 1.0 / (m2 + 1e-8)
    inv3 = 1.0 / (m3 + 1e-8)
    sw = inv1 + inv2 + inv3

    def _comp(c):
        lo, hi = c * 128, (c + 1) * 128
        acc = (inv1 * jnp.where(el1, p1[:, lo:hi], zero)
               + inv2 * jnp.where(el2, p2[:, lo:hi], zero)
               + inv3 * jnp.where(el3, p3[:, lo:hi], zero))
        return jnp.sum(acc, axis=1, keepdims=True)

    out_ref[...] = jnp.concatenate([_comp(0), _comp(1), _comp(2)], axis=1) / sw


def kernel(query_points, ref_points, ref_flow, k):
    del k
    pad = _M_PAD - _M
    rpt = jnp.pad(ref_points, ((0, pad), (0, 0)), constant_values=_FAR).T
    rfa = (jnp.pad(ref_flow, ((0, pad), (0, 0)))
           .reshape(_M_PAD // 128, 128, 3)
           .transpose(0, 2, 1)
           .reshape(_M_PAD // 128, 384))
    rfa = jnp.pad(rfa, ((0, 128 - _M_PAD // 128), (0, 0)))
    rb = rpt.astype(jnp.bfloat16)

    grid = _N // _BQ
    out = pl.pallas_call(
        _knn_flow_kernel,
        grid=(grid,),
        in_specs=[
            pl.BlockSpec((_BQ, 3), lambda b: (b, 0)),
            pl.BlockSpec((3, _M_PAD), lambda b: (0, 0)),
            pl.BlockSpec((3, _M_PAD), lambda b: (0, 0)),
            pl.BlockSpec((128, 384), lambda b: (0, 0)),
        ],
        out_specs=pl.BlockSpec((_BQ, 3), lambda b: (b, 0)),
        out_shape=jax.ShapeDtypeStruct((_N, 3), jnp.float32),
        compiler_params=pltpu.CompilerParams(
            dimension_semantics=("parallel",)),
    )(query_points, rpt, rb, rfa)
    return out


# speedup vs baseline: 1.8016x; 1.8016x over previous
import jax
import jax.numpy as jnp
from jax.experimental import pallas as pl
from jax.experimental.pallas import tpu as pltpu

_N = 10000
_M = 10000
_BQ = 1000
_M_PAD = 10112
_FAR = 1e9


def _knn_flow_kernel(q_ref, rpt_ref, rft_ref, out_ref):
    q = q_ref[...]
    qx, qy, qz = q[:, 0:1], q[:, 1:2], q[:, 2:3]
    rx, ry, rz = rpt_ref[0:1, :], rpt_ref[1:2, :], rpt_ref[2:3, :]

    q2 = qx * qx + qy * qy + qz * qz
    r2 = rx * rx + ry * ry + rz * rz
    rb3 = jnp.concatenate([rx, ry, rz], axis=0).astype(jnp.bfloat16)
    qrm2 = jax.lax.dot_general(
        (q * -2.0).astype(jnp.bfloat16), rb3,
        dimension_numbers=(((1,), (0,)), ((), ())),
        preferred_element_type=jnp.float32,
    )

    big = jnp.float32(jnp.inf)
    zero = jnp.float32(0.0)
    ngrp = _M_PAD // 128
    shape128 = (q.shape[0], 128)
    V1 = jnp.full(shape128, big)
    V2 = jnp.full(shape128, big)
    V3 = jnp.full(shape128, big)
    G1 = jnp.zeros(shape128)
    G2 = jnp.zeros(shape128)
    G3 = jnp.zeros(shape128)
    for g in range(ngrp):
        sl = slice(g * 128, (g + 1) * 128)
        v = jnp.maximum((q2 + qrm2[:, sl]) + r2[:, sl], 0.0)
        gf = jnp.float32(g)
        lt1 = v < V1
        nV1 = jnp.where(lt1, v, V1)
        d1 = jnp.where(lt1, V1, v)
        nG1 = jnp.where(lt1, gf, G1)
        dg1 = jnp.where(lt1, G1, gf)
        lt2 = d1 < V2
        nV2 = jnp.where(lt2, d1, V2)
        d2c = jnp.where(lt2, V2, d1)
        nG2 = jnp.where(lt2, dg1, G2)
        dg2 = jnp.where(lt2, G2, dg1)
        lt3 = d2c < V3
        V3 = jnp.where(lt3, d2c, V3)
        G3 = jnp.where(lt3, dg2, G3)
        V1, V2, G1, G2 = nV1, nV2, nG1, nG2

    iotaL = jax.lax.broadcasted_iota(jnp.int32, (1, 128), 1).astype(jnp.float32)
    bigidx = jnp.float32(1e9)

    def _extract(V1, V2, V3, G1, G2, G3):
        m = jnp.min(V1, axis=1, keepdims=True)
        gidx = G1 * 128.0 + iotaL
        i = jnp.min(jnp.where(V1 == m, gidx, bigidx), axis=1, keepdims=True)
        l = i - 128.0 * jnp.floor(i * (1.0 / 128.0))
        eqL = iotaL == l
        V1 = jnp.where(eqL, V2, V1)
        G1 = jnp.where(eqL, G2, G1)
        V2 = jnp.where(eqL, V3, V2)
        G2 = jnp.where(eqL, G3, G2)
        return m, i, V1, V2, G1, G2

    m1, i1, V1, V2, G1, G2 = _extract(V1, V2, V3, G1, G2, G3)
    m2, i2, V1, V2, G1, G2 = _extract(V1, V2, V3, G1, G2, G3)
    m3, i3, _, _, _, _ = _extract(V1, V2, V3, G1, G2, G3)

    iota128 = jax.lax.broadcasted_iota(jnp.int32, (1, 128), 1).astype(jnp.float32)
    rfa = rft_ref[...]
    zero = jnp.float32(0.0)

    def _pick(i):
        g = jnp.floor(i * (1.0 / 128.0))
        l = i - g * 128.0
        oh = jnp.where(iota128 == g, 1.0, zero)
        p = jax.lax.dot_general(
            oh, rfa, dimension_numbers=(((1,), (0,)), ((), ())),
            preferred_element_type=jnp.float32,
        )
        eql = iota128 == l
        return p, eql

    p1, el1 = _pick(i1)
    p2, el2 = _pick(i2)
    p3, el3 = _pick(i3)

    inv1 = 1.0 / (jnp.sqrt(m1) + 1e-8)
    inv2 = 1.0 / (jnp.sqrt(m2) + 1e-8)
    inv3 = 1.0 / (jnp.sqrt(m3) + 1e-8)
    sw = inv1 + inv2 + inv3

    def _comp(c):
        lo, hi = c * 128, (c + 1) * 128
        acc = (inv1 * jnp.where(el1, p1[:, lo:hi], zero)
               + inv2 * jnp.where(el2, p2[:, lo:hi], zero)
               + inv3 * jnp.where(el3, p3[:, lo:hi], zero))
        return jnp.sum(acc, axis=1, keepdims=True)

    out_ref[...] = jnp.concatenate([_comp(0), _comp(1), _comp(2)], axis=1) / sw


def kernel(query_points, ref_points, ref_flow, k):
    del k
    pad = _M_PAD - _M
    rpt = jnp.pad(ref_points, ((0, pad), (0, 0)), constant_values=_FAR).T
    rfa = (jnp.pad(ref_flow, ((0, pad), (0, 0)))
           .reshape(_M_PAD // 128, 128, 3)
           .transpose(0, 2, 1)
           .reshape(_M_PAD // 128, 384))
    rfa = jnp.pad(rfa, ((0, 128 - _M_PAD // 128), (0, 0)))

    grid = _N // _BQ
    out = pl.pallas_call(
        _knn_flow_kernel,
        grid=(grid,),
        in_specs=[
            pl.BlockSpec((_BQ, 3), lambda b: (b, 0)),
            pl.BlockSpec((3, _M_PAD), lambda b: (0, 0)),
            pl.BlockSpec((128, 384), lambda b: (0, 0)),
        ],
        out_specs=pl.BlockSpec((_BQ, 3), lambda b: (b, 0)),
        out_shape=jax.ShapeDtypeStruct((_N, 3), jnp.float32),
        compiler_params=pltpu.CompilerParams(
            dimension_semantics=("parallel",)),
    )(query_points, rpt, rfa)
    return out
